# Initial kernel scaffold; baseline (speedup 1.0000x reference)
import jax
import jax.numpy as jnp
from jax.experimental import pallas as pl


def kernel(coords, feats, normals, gcn_w1, gcn_w2, gcn_w3, ppf_w0, ppf_w1, ppf_w2, fused_w0, fused_b0, fused_w1, fused_b1):
    raise NotImplementedError("write your pallas kernel here")


# speedup vs baseline: 5.6467x; 5.6467x over previous
import functools

import jax
import jax.numpy as jnp
from jax import lax
from jax.experimental import pallas as pl
from jax.experimental.pallas import tpu as pltpu
from jax.experimental.pallas import tpu_sc as plsc

EPS = 1e-5
K = 32
R2 = 0.3 * 0.3
N = 4096
NK = N * K
NW = 32


def _sel_body(pb_ref, pt_ref, out_ref, d_ref):
    pb = pb_ref[...]
    pt = pt_ref[...]
    srow = jnp.sum(pb * pb, axis=1, keepdims=True)
    scol = jnp.sum(pt * pt, axis=0, keepdims=True)
    D = srow + scol - 2.0 * jnp.dot(pb, pt, preferred_element_type=jnp.float32)

    iot = lax.broadcasted_iota(jnp.int32, (128, N), 1)
    col = lax.broadcasted_iota(jnp.int32, (128, 128), 1)

    mask = D <= R2
    cnt = mask.astype(jnp.int32)
    sh = 1
    while sh < N:
        cnt = cnt + jnp.concatenate(
            [jnp.zeros((128, sh), jnp.int32), cnt[:, : N - sh]], axis=1)
        sh *= 2
    ball0 = jnp.min(jnp.where(mask & (cnt == 1), iot, N), axis=1)
    res = jnp.where(col == 64, ball0[:, None], jnp.zeros((128, 128), jnp.int32))

    def ball_step(s, res):
        cand = jnp.where(mask & (cnt == s + 1), iot, N)
        idx = jnp.min(cand, axis=1)
        idx = jnp.where(idx == N, ball0, idx)
        return jnp.where(col == 64 + s, idx[:, None], res)

    res = lax.fori_loop(1, K, ball_step, res)

    d_ref[...] = D

    def topk_step(t, res):
        Dw = d_ref[...]
        m = jnp.min(Dw, axis=1, keepdims=True)
        arg = jnp.min(jnp.where(Dw == m, iot, N), axis=1)
        d_ref[...] = jnp.where(iot == arg[:, None], jnp.float32(jnp.inf), Dw)
        return jnp.where(col == t, arg[:, None], res)

    res = lax.fori_loop(0, K + 1, topk_step, res)
    out_ref[...] = res


def _select(ppad, ppad_t):
    return pl.pallas_call(
        _sel_body,
        grid=(N // 128,),
        in_specs=[
            pl.BlockSpec((128, 8), lambda i: (i, 0)),
            pl.BlockSpec((8, N), lambda i: (0, 0)),
        ],
        out_specs=pl.BlockSpec((128, 128), lambda i: (i, 0)),
        out_shape=jax.ShapeDtypeStruct((N, 128), jnp.int32),
        scratch_shapes=[pltpu.VMEM((128, N), jnp.float32)],
    )(ppad, ppad_t)


def _sc_gather(table, idxflat):
    C = table.shape[1]
    rows_pw = NK // NW
    chunk = min(rows_pw, max(8, 65536 // C))
    nchunks = rows_pw // chunk
    mesh = plsc.VectorSubcoreMesh(core_axis_name="c", subcore_axis_name="s")

    @functools.partial(
        pl.kernel,
        mesh=mesh,
        out_type=jax.ShapeDtypeStruct((NK, C), jnp.float32),
        scratch_types=[
            pltpu.VMEM((chunk,), jnp.int32),
            pltpu.VMEM((chunk, C), jnp.float32),
            pltpu.SemaphoreType.DMA,
        ],
    )
    def k(table_hbm, idx_hbm, out_hbm, idx_v, rows_v, sem):
        wid = lax.axis_index("s") * 2 + lax.axis_index("c")
        base = wid * rows_pw
        for j in range(nchunks):
            off = base + j * chunk
            pltpu.sync_copy(idx_hbm.at[pl.ds(off, chunk)], idx_v)
            pltpu.async_copy(table_hbm.at[idx_v], rows_v, sem).wait()
            pltpu.sync_copy(rows_v, out_hbm.at[pl.ds(off, chunk)])

    return k(table, idxflat)


def _sc_ppf_gather(cols, ballflat):
    rows_pw = NK // NW
    mesh = plsc.VectorSubcoreMesh(core_axis_name="c", subcore_axis_name="s")

    @functools.partial(
        pl.kernel,
        mesh=mesh,
        out_type=jax.ShapeDtypeStruct((8, NK), jnp.float32),
        scratch_types=[
            pltpu.VMEM((rows_pw,), jnp.int32),
            pltpu.VMEM((rows_pw,), jnp.float32),
            pltpu.SemaphoreType.DMA,
        ],
    )
    def k(c0, c1, c2, c3, c4, c5, idx_hbm, out_hbm, idx_v, val_v, sem):
        wid = lax.axis_index("s") * 2 + lax.axis_index("c")
        base = wid * rows_pw
        pltpu.sync_copy(idx_hbm.at[pl.ds(base, rows_pw)], idx_v)
        for c, tbl in enumerate((c0, c1, c2, c3, c4, c5)):
            pltpu.async_copy(tbl.at[idx_v], val_v, sem).wait()
            pltpu.sync_copy(val_v, out_hbm.at[c, pl.ds(base, rows_pw)])

    return k(*cols, ballflat)


def _ppf_ang_body(gp_ref, pt_ref, nt_ref, w0_ref, f10_ref, st_ref):
    kstep = pl.program_id(0)
    gp = gp_ref[...]
    px, py, pz = pt_ref[0:1, :], pt_ref[1:2, :], pt_ref[2:3, :]
    nix, niy, niz = nt_ref[0:1, :], nt_ref[1:2, :], nt_ref[2:3, :]
    gx = gp[0:1, :] - px
    gy = gp[1:2, :] - py
    gz = gp[2:3, :] - pz
    njx, njy, njz = gp[3:4, :], gp[4:5, :], gp[5:6, :]

    def ang(ax, ay, az, bx, by, bz):
        cx = ay * bz - az * by
        cy = az * bx - ax * bz
        cz = ax * by - ay * bx
        yv = jnp.sqrt(cx * cx + cy * cy + cz * cz + 1e-12)
        xv = ax * bx + ay * by + az * bz
        return jnp.arctan2(yv, xv)

    nr_d = ang(nix, niy, niz, gx, gy, gz)
    ni_d = ang(njx, njy, njz, gx, gy, gz)
    nr_ni = ang(nix, niy, niz, njx, njy, njz)
    dn = jnp.sqrt(gx * gx + gy * gy + gz * gz + 1e-12)
    zr = jnp.zeros((6, N), jnp.float32)
    f10 = jnp.concatenate(
        [px, py, pz, gx, gy, gz, nr_d, ni_d, nr_ni, dn, zr], axis=0)
    f10_ref[...] = f10

    x0 = jnp.dot(w0_ref[...], f10, preferred_element_type=jnp.float32)

    @pl.when(kstep == 0)
    def _():
        st_ref[...] = jnp.zeros_like(st_ref)

    st_ref[0:1, 0:64] += jnp.sum(x0, axis=1)[None, :]
    st_ref[1:2, 0:64] += jnp.sum(x0 * x0, axis=1)[None, :]


def _ppf_ang(gplanes, pt, nt, w0p):
    return pl.pallas_call(
        _ppf_ang_body,
        grid=(K,),
        in_specs=[
            pl.BlockSpec((8, N), lambda k: (0, k)),
            pl.BlockSpec((4, N), lambda k: (0, 0)),
            pl.BlockSpec((4, N), lambda k: (0, 0)),
            pl.BlockSpec((64, 16), lambda k: (0, 0)),
        ],
        out_specs=[
            pl.BlockSpec((16, N), lambda k: (0, k)),
            pl.BlockSpec((8, 128), lambda k: (0, 0)),
        ],
        out_shape=[
            jax.ShapeDtypeStruct((16, NK), jnp.float32),
            jax.ShapeDtypeStruct((8, 128), jnp.float32),
        ],
    )(gplanes, pt, nt, w0p)


def _ppf_mid_body(f10_ref, w0_ref, w1_ref, m0_ref, i0_ref, st_ref):
    kstep = pl.program_id(0)
    x0 = jnp.dot(w0_ref[...], f10_ref[...], preferred_element_type=jnp.float32)
    h0 = (x0 - m0_ref[...]) * i0_ref[...]
    h0 = jnp.maximum(h0, 0.0)
    x1 = jnp.dot(w1_ref[...], h0, preferred_element_type=jnp.float32)

    @pl.when(kstep == 0)
    def _():
        st_ref[...] = jnp.zeros_like(st_ref)

    st_ref[0:1, :] += jnp.sum(x1, axis=1)[None, :]
    st_ref[1:2, :] += jnp.sum(x1 * x1, axis=1)[None, :]


def _ppf_mid(f10, w0p, w1, m0, i0):
    return pl.pallas_call(
        _ppf_mid_body,
        grid=(K,),
        in_specs=[
            pl.BlockSpec((16, N), lambda k: (0, k)),
            pl.BlockSpec((64, 16), lambda k: (0, 0)),
            pl.BlockSpec((128, 64), lambda k: (0, 0)),
            pl.BlockSpec((64, 1), lambda k: (0, 0)),
            pl.BlockSpec((64, 1), lambda k: (0, 0)),
        ],
        out_specs=pl.BlockSpec((8, 128), lambda k: (0, 0)),
        out_shape=jax.ShapeDtypeStruct((8, 128), jnp.float32),
    )(f10, w0p, w1, m0, i0)


def _ppf_last_body(f10_ref, w0_ref, w1_ref, w2_ref, m0_ref, i0_ref,
                   m1_ref, i1_ref, xmax_ref, st_ref):
    kstep = pl.program_id(0)
    x0 = jnp.dot(w0_ref[...], f10_ref[...], preferred_element_type=jnp.float32)
    h0 = jnp.maximum((x0 - m0_ref[...]) * i0_ref[...], 0.0)
    x1 = jnp.dot(w1_ref[...], h0, preferred_element_type=jnp.float32)
    h1 = jnp.maximum((x1 - m1_ref[...]) * i1_ref[...], 0.0)
    x2 = jnp.dot(w2_ref[...], h1, preferred_element_type=jnp.float32)

    @pl.when(kstep == 0)
    def _():
        xmax_ref[...] = x2
        st_ref[...] = jnp.zeros_like(st_ref)

    @pl.when(kstep > 0)
    def _():
        xmax_ref[...] = jnp.maximum(xmax_ref[...], x2)

    st_ref[0:1, 0:64] += jnp.sum(x2, axis=1)[None, :]
    st_ref[1:2, 0:64] += jnp.sum(x2 * x2, axis=1)[None, :]


def _ppf_last(f10, w0p, w1, w2, m0, i0, m1, i1):
    return pl.pallas_call(
        _ppf_last_body,
        grid=(K,),
        in_specs=[
            pl.BlockSpec((16, N), lambda k: (0, k)),
            pl.BlockSpec((64, 16), lambda k: (0, 0)),
            pl.BlockSpec((128, 64), lambda k: (0, 0)),
            pl.BlockSpec((64, 128), lambda k: (0, 0)),
            pl.BlockSpec((64, 1), lambda k: (0, 0)),
            pl.BlockSpec((64, 1), lambda k: (0, 0)),
            pl.BlockSpec((128, 1), lambda k: (0, 0)),
            pl.BlockSpec((128, 1), lambda k: (0, 0)),
        ],
        out_specs=[
            pl.BlockSpec((64, N), lambda k: (0, 0)),
            pl.BlockSpec((8, 128), lambda k: (0, 0)),
        ],
        out_shape=[
            jax.ShapeDtypeStruct((64, N), jnp.float32),
            jax.ShapeDtypeStruct((8, 128), jnp.float32),
        ],
    )(f10, w0p, w1, w2, m0, i0, m1, i1)


def _mm2_body(f_ref, wa_ref, wb_ref, ya_ref, yb_ref):
    f = f_ref[...]
    ya_ref[...] = jnp.dot(f, wa_ref[...], preferred_element_type=jnp.float32)
    yb_ref[...] = jnp.dot(f, wb_ref[...], preferred_element_type=jnp.float32)


def _mm2(f, wa, wb):
    Cin = f.shape[1]
    Ca, Cb = wa.shape[1], wb.shape[1]
    return pl.pallas_call(
        _mm2_body,
        grid=(N // 512,),
        in_specs=[
            pl.BlockSpec((512, Cin), lambda i: (i, 0)),
            pl.BlockSpec((Cin, Ca), lambda i: (0, 0)),
            pl.BlockSpec((Cin, Cb), lambda i: (0, 0)),
        ],
        out_specs=[
            pl.BlockSpec((512, Ca), lambda i: (i, 0)),
            pl.BlockSpec((512, Cb), lambda i: (i, 0)),
        ],
        out_shape=[
            jax.ShapeDtypeStruct((N, Ca), jnp.float32),
            jax.ShapeDtypeStruct((N, Cb), jnp.float32),
        ],
    )(f, wa, wb)


def _ecred_body(g_ref, z_ref, m_ref, s1_ref, s2_ref, st_ref):
    kstep = pl.program_id(0)
    g = g_ref[...]

    @pl.when(kstep == 0)
    def _():
        m_ref[...] = g
        s1_ref[...] = g
        s2_ref[...] = g * g

    @pl.when(kstep > 0)
    def _():
        m_ref[...] = jnp.maximum(m_ref[...], g)
        s1_ref[...] = s1_ref[...] + g
        s2_ref[...] = s2_ref[...] + g * g

    @pl.when(kstep == K - 1)
    def _():
        z = z_ref[...]
        s1 = s1_ref[...]
        s2 = s2_ref[...]
        tot = jnp.sum(K * z + s1, axis=0)
        totsq = jnp.sum(K * z * z + 2.0 * z * s1 + s2, axis=0)
        st_ref[0:1, :] = tot[None, :]
        st_ref[1:2, :] = totsq[None, :]


def _ecred(g, z):
    C = z.shape[1]
    return pl.pallas_call(
        _ecred_body,
        grid=(K,),
        in_specs=[
            pl.BlockSpec((N, C), lambda k: (k, 0)),
            pl.BlockSpec((N, C), lambda k: (0, 0)),
        ],
        out_specs=[
            pl.BlockSpec((N, C), lambda k: (0, 0)),
            pl.BlockSpec((N, C), lambda k: (0, 0)),
            pl.BlockSpec((N, C), lambda k: (0, 0)),
            pl.BlockSpec((8, C), lambda k: (0, 0)),
        ],
        out_shape=[
            jax.ShapeDtypeStruct((N, C), jnp.float32),
            jax.ShapeDtypeStruct((N, C), jnp.float32),
            jax.ShapeDtypeStruct((N, C), jnp.float32),
            jax.ShapeDtypeStruct((8, C), jnp.float32),
        ],
    )(g, z)


def _lrelu(x):
    return jnp.where(x >= 0, x, 0.2 * x)


def _gcnb_body(z_ref, mx_ref, mu_ref, iv_ref, wa_ref, wb_ref,
               f1_ref, ya_ref, yb_ref):
    f1 = _lrelu((z_ref[...] + mx_ref[...] - mu_ref[...]) * iv_ref[...])
    f1_ref[...] = f1
    ya_ref[...] = jnp.dot(f1, wa_ref[...], preferred_element_type=jnp.float32)
    yb_ref[...] = jnp.dot(f1, wb_ref[...], preferred_element_type=jnp.float32)


def _gcnb(z, mx, mu, iv, wa, wb):
    Cin = z.shape[1]
    Ca, Cb = wa.shape[1], wb.shape[1]
    return pl.pallas_call(
        _gcnb_body,
        grid=(N // 512,),
        in_specs=[
            pl.BlockSpec((512, Cin), lambda i: (i, 0)),
            pl.BlockSpec((512, Cin), lambda i: (i, 0)),
            pl.BlockSpec((1, Cin), lambda i: (0, 0)),
            pl.BlockSpec((1, Cin), lambda i: (0, 0)),
            pl.BlockSpec((Cin, Ca), lambda i: (0, 0)),
            pl.BlockSpec((Cin, Cb), lambda i: (0, 0)),
        ],
        out_specs=[
            pl.BlockSpec((512, Cin), lambda i: (i, 0)),
            pl.BlockSpec((512, Ca), lambda i: (i, 0)),
            pl.BlockSpec((512, Cb), lambda i: (i, 0)),
        ],
        out_shape=[
            jax.ShapeDtypeStruct((N, Cin), jnp.float32),
            jax.ShapeDtypeStruct((N, Ca), jnp.float32),
            jax.ShapeDtypeStruct((N, Cb), jnp.float32),
        ],
    )(z, mx, mu, iv, wa, wb)


def _gcnc_body(f_ref, f1_ref, z2_ref, mx2_ref, mu_ref, iv_ref, w3_ref,
               g3_ref, st_ref):
    i = pl.program_id(0)
    f2 = _lrelu((z2_ref[...] + mx2_ref[...] - mu_ref[...]) * iv_ref[...])
    f3 = jnp.concatenate([f_ref[...], f1_ref[...], f2], axis=1)
    g3 = jnp.dot(f3, w3_ref[...], preferred_element_type=jnp.float32)
    g3_ref[...] = g3

    @pl.when(i == 0)
    def _():
        st_ref[...] = jnp.zeros_like(st_ref)

    st_ref[0:1, :] += jnp.sum(g3, axis=0)[None, :]
    st_ref[1:2, :] += jnp.sum(g3 * g3, axis=0)[None, :]


def _gcnc(f, f1, z2, mx2, mu, iv, w3t):
    return pl.pallas_call(
        _gcnc_body,
        grid=(N // 512,),
        in_specs=[
            pl.BlockSpec((512, 128), lambda i: (i, 0)),
            pl.BlockSpec((512, 128), lambda i: (i, 0)),
            pl.BlockSpec((512, 256), lambda i: (i, 0)),
            pl.BlockSpec((512, 256), lambda i: (i, 0)),
            pl.BlockSpec((1, 256), lambda i: (0, 0)),
            pl.BlockSpec((1, 256), lambda i: (0, 0)),
            pl.BlockSpec((512, 128), lambda i: (0, 0)),
        ],
        out_specs=[
            pl.BlockSpec((512, 128), lambda i: (i, 0)),
            pl.BlockSpec((8, 128), lambda i: (0, 0)),
        ],
        out_shape=[
            jax.ShapeDtypeStruct((N, 128), jnp.float32),
            jax.ShapeDtypeStruct((8, 128), jnp.float32),
        ],
    )(f, f1, z2, mx2, mu, iv, w3t)


def _heada_body(g3_ref, xm_ref, m3_ref, i3_ref, mp_ref, ip_ref,
                w0_ref, b0_ref, h0_ref, st_ref):
    i = pl.program_id(0)
    fgcn = _lrelu((g3_ref[...] - m3_ref[...]) * i3_ref[...])
    fppf = jnp.maximum((xm_ref[...] - mp_ref[...]) * ip_ref[...], 0.0)
    h = jnp.concatenate([fppf, fgcn], axis=1)
    h0 = jnp.dot(h, w0_ref[...], preferred_element_type=jnp.float32) + b0_ref[...]
    h0_ref[...] = h0

    @pl.when(i == 0)
    def _():
        st_ref[...] = jnp.zeros_like(st_ref)

    st_ref[0:1, :] += jnp.sum(h0, axis=0)[None, :]
    st_ref[1:2, :] += jnp.sum(h0 * h0, axis=0)[None, :]


def _heada(g3, xmt, m3, i3, mp, ip, w0t, b0):
    return pl.pallas_call(
        _heada_body,
        grid=(N // 512,),
        in_specs=[
            pl.BlockSpec((512, 128), lambda i: (i, 0)),
            pl.BlockSpec((512, 64), lambda i: (i, 0)),
            pl.BlockSpec((1, 128), lambda i: (0, 0)),
            pl.BlockSpec((1, 128), lambda i: (0, 0)),
            pl.BlockSpec((1, 64), lambda i: (0, 0)),
            pl.BlockSpec((1, 64), lambda i: (0, 0)),
            pl.BlockSpec((192, 192), lambda i: (0, 0)),
            pl.BlockSpec((1, 192), lambda i: (0, 0)),
        ],
        out_specs=[
            pl.BlockSpec((512, 192), lambda i: (i, 0)),
            pl.BlockSpec((8, 192), lambda i: (0, 0)),
        ],
        out_shape=[
            jax.ShapeDtypeStruct((N, 192), jnp.float32),
            jax.ShapeDtypeStruct((8, 192), jnp.float32),
        ],
    )(g3, xmt, m3, i3, mp, ip, w0t, b0)


def _headb_body(h0_ref, m_ref, iv_ref, w1_ref, b1_ref, h1_ref, st_ref):
    i = pl.program_id(0)
    a0 = _lrelu((h0_ref[...] - m_ref[...]) * iv_ref[...])
    h1 = jnp.dot(a0, w1_ref[...], preferred_element_type=jnp.float32) + b1_ref[...]
    h1_ref[...] = h1

    @pl.when(i == 0)
    def _():
        st_ref[...] = jnp.zeros_like(st_ref)

    st_ref[0:1, :] += jnp.sum(h1, axis=0)[None, :]
    st_ref[1:2, :] += jnp.sum(h1 * h1, axis=0)[None, :]


def _headb(h0, m, iv, w1t, b1):
    return pl.pallas_call(
        _headb_body,
        grid=(N // 512,),
        in_specs=[
            pl.BlockSpec((512, 192), lambda i: (i, 0)),
            pl.BlockSpec((1, 192), lambda i: (0, 0)),
            pl.BlockSpec((1, 192), lambda i: (0, 0)),
            pl.BlockSpec((192, 128), lambda i: (0, 0)),
            pl.BlockSpec((1, 128), lambda i: (0, 0)),
        ],
        out_specs=[
            pl.BlockSpec((512, 128), lambda i: (i, 0)),
            pl.BlockSpec((8, 128), lambda i: (0, 0)),
        ],
        out_shape=[
            jax.ShapeDtypeStruct((N, 128), jnp.float32),
            jax.ShapeDtypeStruct((8, 128), jnp.float32),
        ],
    )(h0, m, iv, w1t, b1)


def _headc_body(h1_ref, m_ref, iv_ref, o_ref):
    o_ref[...] = _lrelu((h1_ref[...] - m_ref[...]) * iv_ref[...])


def _headc(h1, m, iv):
    return pl.pallas_call(
        _headc_body,
        grid=(N // 512,),
        in_specs=[
            pl.BlockSpec((512, 128), lambda i: (i, 0)),
            pl.BlockSpec((1, 128), lambda i: (0, 0)),
            pl.BlockSpec((1, 128), lambda i: (0, 0)),
        ],
        out_specs=pl.BlockSpec((512, 128), lambda i: (i, 0)),
        out_shape=jax.ShapeDtypeStruct((N, 128), jnp.float32),
    )(h1, m, iv)


def _mi(st, count, C, rowvec):
    s = st[0, :C]
    sq = st[1, :C]
    m = s / count
    v = sq / count - m * m
    iv = lax.rsqrt(v + EPS)
    if rowvec:
        return m[None, :], iv[None, :]
    return m[:, None], iv[:, None]


def kernel(coords, feats, normals, gcn_w1, gcn_w2, gcn_w3, ppf_w0, ppf_w1,
           ppf_w2, fused_w0, fused_b0, fused_w1, fused_b1):
    P = coords[0].T
    F = feats[0].T
    Nm = normals[0].T

    ppad = jnp.pad(P, ((0, 0), (0, 5)))
    ppad_t = ppad.T
    sel = _select(ppad, ppad_t)
    knn_flat = sel[:, 1:K + 1].T.reshape(NK)
    ball_flat = sel[:, 64:64 + K].T.reshape(NK)

    cols = (P[:, 0], P[:, 1], P[:, 2], Nm[:, 0], Nm[:, 1], Nm[:, 2])
    gplanes = _sc_ppf_gather(cols, ball_flat)
    pt4 = jnp.pad(P.T, ((0, 1), (0, 0)))
    nt4 = jnp.pad(Nm.T, ((0, 1), (0, 0)))
    w0p = jnp.pad(ppf_w0, ((0, 0), (0, 6)))
    f10, st0 = _ppf_ang(gplanes, pt4, nt4, w0p)
    m0, i0 = _mi(st0, NK, 64, rowvec=False)
    st1 = _ppf_mid(f10, w0p, ppf_w1, m0, i0)
    m1, i1 = _mi(st1, NK, 128, rowvec=False)
    xmax, st2 = _ppf_last(f10, w0p, ppf_w1, ppf_w2, m0, i0, m1, i1)
    mp, ip = _mi(st2, NK, 64, rowvec=True)

    wc1, wn1 = gcn_w1[:, :128], gcn_w1[:, 128:]
    y1, z1 = _mm2(F, wn1.T, (wc1 - wn1).T)
    g1 = _sc_gather(y1, knn_flat)
    mx1, _s11, _s21, stE1 = _ecred(g1, z1)
    mu1, iv1 = _mi(stE1, NK, 128, rowvec=True)

    wc2, wn2 = gcn_w2[:, :128], gcn_w2[:, 128:]
    f1, y2, z2 = _gcnb(z1, mx1, mu1, iv1, wn2.T, (wc2 - wn2).T)
    g2 = _sc_gather(y2, knn_flat)
    mx2, _s12, _s22, stE2 = _ecred(g2, z2)
    mu2, iv2 = _mi(stE2, NK, 256, rowvec=True)

    g3, st3 = _gcnc(F, f1, z2, mx2, mu2, iv2, gcn_w3.T)
    m3, i3 = _mi(st3, N, 128, rowvec=True)

    h0, stH0 = _heada(g3, xmax.T, m3, i3, mp, ip, fused_w0.T,
                      fused_b0[None, :])
    mh0, ih0 = _mi(stH0, N, 192, rowvec=True)
    h1, stH1 = _headb(h0, mh0, ih0, fused_w1.T, fused_b1[None, :])
    mh1, ih1 = _mi(stH1, N, 128, rowvec=True)
    out = _headc(h1, mh1, ih1)
    return out.T[None]
